# baseline (speedup 1.0000x reference)
import jax
import jax.numpy as jnp
from jax import lax
from jax.experimental import pallas as pl
from jax.experimental.pallas import tpu as pltpu
from jax.experimental.pallas import tpu_sc as plsc

_B, _C, _N = 16, 256, 16384
_K = 4
_ROWS = _B * _C
_SC_ROWS = 1280
_TC_ROWS = _ROWS - _SC_ROWS
_NC, _NS = 2, 16
_NW = _NC * _NS
_X2_ROWS = _ROWS * _N // 128
_SC_IN_R = _SC_ROWS * _N // 128
_SC_OUT_R = _SC_IN_R // _K
_R_PER_W = _SC_IN_R // _NW
_CH_R = 256
_CH_OR = _CH_R // _K
_N_CHUNKS = _R_PER_W // _CH_R
_TC_BLK = 128


def _sc_pool_body(x_hbm, out_hbm, xv0, xv1, ov0, ov1, is0, is1, os0, os1):
    wid = lax.axis_index("s") * _NC + lax.axis_index("c")
    in_base = wid * _R_PER_W
    out_base = wid * (_R_PER_W // _K)
    lane = lax.broadcasted_iota(jnp.int32, (16,), 0)
    perm1 = lane ^ 1
    rep_a = (lane & 3) * 4
    rep_b = rep_a + 2
    quarter = lane >> 2
    masks = [quarter == w for w in range(3)]
    xvs, ovs = (xv0, xv1), (ov0, ov1)
    isems, osems = (is0, is1), (os0, os1)

    pltpu.async_copy(x_hbm.at[pl.ds(in_base, _CH_R)], xv0, is0)

    def pair_body(c0, carry):
        for b in (0, 1):
            c = 2 * c0 + b
            pltpu.make_async_copy(
                x_hbm.at[pl.ds(0, _CH_R)], xvs[b], isems[b]).wait()

            @pl.when(c + 1 < _N_CHUNKS)
            def _():
                pltpu.async_copy(
                    x_hbm.at[pl.ds(in_base + (c + 1) * _CH_R, _CH_R)],
                    xvs[1 - b], isems[1 - b])

            @pl.when(c0 >= 1)
            def _():
                pltpu.make_async_copy(
                    ovs[b], out_hbm.at[pl.ds(0, _CH_OR)], osems[b]).wait()

            x_v, out_v = xvs[b], ovs[b]

            @plsc.parallel_loop(0, _CH_OR, unroll=2)
            def _(o):
                for q in range(4):
                    cs = []
                    for w in range(8):
                        v = x_v[4 * o + q, pl.ds(w * 16, 16)]
                        s1 = v + jnp.take_along_axis(v, perm1, axis=0)
                        c_w = (jnp.take_along_axis(s1, rep_a, axis=0)
                               + jnp.take_along_axis(s1, rep_b, axis=0))
                        cs.append(c_w)
                    for h in (0, 1):
                        g = cs[4 * h:4 * h + 4]
                        merged = jnp.where(
                            masks[0], g[0],
                            jnp.where(masks[1], g[1],
                                      jnp.where(masks[2], g[2], g[3])))
                        out_v[o, pl.ds(q * 32 + h * 16, 16)] = merged * 0.25

            pltpu.async_copy(
                ovs[b], out_hbm.at[pl.ds(out_base + c * _CH_OR, _CH_OR)],
                osems[b])
        return carry

    lax.fori_loop(0, _N_CHUNKS // 2, pair_body, 0)
    for b in (0, 1):
        pltpu.make_async_copy(
            ovs[b], out_hbm.at[pl.ds(0, _CH_OR)], osems[b]).wait()


def _sc_pool(x2):
    mesh = plsc.VectorSubcoreMesh(core_axis_name="c", subcore_axis_name="s")
    return pl.kernel(
        _sc_pool_body,
        out_type=jax.ShapeDtypeStruct((_SC_OUT_R, 128), jnp.float32),
        mesh=mesh,
        scratch_types=[
            pltpu.VMEM((_CH_R, 128), jnp.float32),
            pltpu.VMEM((_CH_R, 128), jnp.float32),
            pltpu.VMEM((_CH_OR, 128), jnp.float32),
            pltpu.VMEM((_CH_OR, 128), jnp.float32),
            pltpu.SemaphoreType.DMA,
            pltpu.SemaphoreType.DMA,
            pltpu.SemaphoreType.DMA,
            pltpu.SemaphoreType.DMA,
        ],
        compiler_params=pltpu.CompilerParams(needs_layout_passes=False),
    )(x2)


def _tc_pool_kernel(x_ref, o_ref):
    blk = x_ref[...]
    xb = blk.reshape(_TC_BLK * 128, 128)
    l = lax.broadcasted_iota(jnp.int32, (128, 32), 0)
    m = lax.broadcasted_iota(jnp.int32, (128, 32), 1)
    p = jnp.where((l >> 2) == m, 0.25, 0.0).astype(jnp.float32)
    y = lax.dot_general(xb, p, (((1,), (0,)), ((), ())),
                        preferred_element_type=jnp.float32)
    o_ref[...] = y.reshape(_TC_BLK, 128, 32)


def _tc_pool(x3):
    base = _SC_ROWS // _TC_BLK
    return pl.pallas_call(
        _tc_pool_kernel,
        grid=(_TC_ROWS // _TC_BLK,),
        in_specs=[pl.BlockSpec((_TC_BLK, 128, 128),
                               lambda i: (base + i, 0, 0))],
        out_specs=pl.BlockSpec((_TC_BLK, 128, 32),
                               lambda i: (base + i, 0, 0)),
        out_shape=jax.ShapeDtypeStruct((_ROWS, 128, 32), jnp.float32),
    )(x3)


def _asm_kernel(tc_ref, sc_ref, o_ref):
    del tc_ref
    o_ref[...] = sc_ref[...]


def _assemble(tc_full, sc_out3):
    return pl.pallas_call(
        _asm_kernel,
        grid=(_SC_ROWS // _TC_BLK,),
        in_specs=[
            pl.BlockSpec(memory_space=pl.ANY),
            pl.BlockSpec((_TC_BLK, 128, 32), lambda i: (i, 0, 0)),
        ],
        out_specs=pl.BlockSpec((_TC_BLK, 128, 32), lambda i: (i, 0, 0)),
        out_shape=jax.ShapeDtypeStruct((_ROWS, 128, 32), jnp.float32),
        input_output_aliases={0: 0},
    )(tc_full, sc_out3)


def kernel(x):
    x3 = x.reshape(_ROWS, 128, 128)
    sc_out = _sc_pool(x.reshape(_X2_ROWS, 128))
    tc_full = _tc_pool(x3)
    out = _assemble(tc_full, sc_out.reshape(_SC_ROWS, 128, 32))
    return out.reshape(_B, _C, _N // _K)


# speedup vs baseline: 1.6502x; 1.6502x over previous
import jax
import jax.numpy as jnp
from jax import lax
from jax.experimental import pallas as pl
from jax.experimental.pallas import tpu as pltpu
from jax.experimental.pallas import tpu_sc as plsc

_B, _C, _N = 16, 256, 16384
_K = 4
_TOTAL_IN = _B * _C * _N
_TOTAL_OUT = _TOTAL_IN // _K
_NC, _NS = 2, 16
_NW = _NC * _NS
_X2_ROWS = _TOTAL_IN // 128
_R_PER_W = _X2_ROWS // _NW
_CH_R = 256
_CH_IN = _CH_R * 128
_CH_OUT = _CH_IN // _K
_N_CHUNKS = _R_PER_W // _CH_R
_VREGS = _CH_IN // 16


def _pool_body(x_hbm, out_hbm, xv0, xv1, ov0, ov1, is0, is1, os0, os1):
    wid = lax.axis_index("s") * _NC + lax.axis_index("c")
    in_base = wid * _R_PER_W
    out_base = wid * (_R_PER_W * 128 // _K)
    lane = lax.broadcasted_iota(jnp.int32, (16,), 0)
    perm1 = lane ^ 1
    perm2 = lane ^ 2
    mask4 = (lane & 3) == 0
    xvs, ovs = (xv0, xv1), (ov0, ov1)
    isems, osems = (is0, is1), (os0, os1)

    pltpu.async_copy(x_hbm.at[pl.ds(in_base, _CH_R)], xv0, is0)

    def pair_body(c0, carry):
        for b in (0, 1):
            c = 2 * c0 + b
            pltpu.make_async_copy(
                x_hbm.at[pl.ds(0, _CH_R)], xvs[b], isems[b]).wait()

            @pl.when(c + 1 < _N_CHUNKS)
            def _():
                pltpu.async_copy(
                    x_hbm.at[pl.ds(in_base + (c + 1) * _CH_R, _CH_R)],
                    xvs[1 - b], isems[1 - b])

            @pl.when(c0 >= 1)
            def _():
                pltpu.make_async_copy(
                    ovs[b].at[pl.ds(0, _CH_OUT)],
                    out_hbm.at[pl.ds(0, _CH_OUT)], osems[b]).wait()

            x_v, out_v = xvs[b], ovs[b]

            @plsc.parallel_loop(0, _VREGS, unroll=8)
            def _(i):
                v = x_v[i >> 3, pl.ds((i & 7) * 16, 16)]
                s1 = v + jnp.take_along_axis(v, perm1, axis=0)
                s2 = s1 + jnp.take_along_axis(s1, perm2, axis=0)
                plsc.store_compressed(
                    out_v.at[pl.ds(i * 4, 16)], s2 * 0.25, mask=mask4)

            pltpu.async_copy(
                out_v.at[pl.ds(0, _CH_OUT)],
                out_hbm.at[pl.ds(out_base + c * _CH_OUT, _CH_OUT)],
                osems[b])
        return carry

    lax.fori_loop(0, _N_CHUNKS // 2, pair_body, 0)
    for b in (0, 1):
        pltpu.make_async_copy(
            ovs[b].at[pl.ds(0, _CH_OUT)],
            out_hbm.at[pl.ds(0, _CH_OUT)], osems[b]).wait()


def kernel(x):
    x2 = x.reshape(_X2_ROWS, 128)
    mesh = plsc.VectorSubcoreMesh(core_axis_name="c", subcore_axis_name="s")
    out = pl.kernel(
        _pool_body,
        out_type=jax.ShapeDtypeStruct((_TOTAL_OUT,), jnp.float32),
        mesh=mesh,
        scratch_types=[
            pltpu.VMEM((_CH_R, 128), jnp.float32),
            pltpu.VMEM((_CH_R, 128), jnp.float32),
            pltpu.VMEM((_CH_OUT + 16,), jnp.float32),
            pltpu.VMEM((_CH_OUT + 16,), jnp.float32),
            pltpu.SemaphoreType.DMA,
            pltpu.SemaphoreType.DMA,
            pltpu.SemaphoreType.DMA,
            pltpu.SemaphoreType.DMA,
        ],
        compiler_params=pltpu.CompilerParams(needs_layout_passes=False),
    )(x2)
    return out.reshape(_B, _C, _N // _K)


# speedup vs baseline: 2.1189x; 1.2841x over previous
import jax
import jax.numpy as jnp
from jax import lax
from jax.experimental import pallas as pl
from jax.experimental.pallas import tpu as pltpu
from jax.experimental.pallas import tpu_sc as plsc

_B, _C, _N = 16, 256, 16384
_K = 4
_TOTAL_IN = _B * _C * _N
_TOTAL_OUT = _TOTAL_IN // _K
_NC, _NS = 2, 16
_NW = _NC * _NS
_IN_PER_W = _TOTAL_IN // _NW
_CH_IN = 32768
_CH_OUT = _CH_IN // _K
_N_CHUNKS = _IN_PER_W // _CH_IN
_GROUPS = _CH_IN // 64


def _pool_body(x_hbm, out_hbm, xv0, xv1, ov0, ov1, is0, is1, os0, os1):
    wid = lax.axis_index("s") * _NC + lax.axis_index("c")
    in_base = wid * _IN_PER_W
    out_base = wid * (_IN_PER_W // _K)
    lane = lax.broadcasted_iota(jnp.int32, (16,), 0)
    idx = [lane * _K + r for r in range(_K)]
    xvs, ovs = (xv0, xv1), (ov0, ov1)
    isems, osems = (is0, is1), (os0, os1)

    pltpu.async_copy(x_hbm.at[pl.ds(in_base, _CH_IN)], xv0, is0)

    def pair_body(c0, carry):
        for b in (0, 1):
            c = 2 * c0 + b
            pltpu.make_async_copy(
                x_hbm.at[pl.ds(0, _CH_IN)], xvs[b], isems[b]).wait()

            @pl.when(c + 1 < _N_CHUNKS)
            def _():
                pltpu.async_copy(
                    x_hbm.at[pl.ds(in_base + (c + 1) * _CH_IN, _CH_IN)],
                    xvs[1 - b], isems[1 - b])

            @pl.when(c0 >= 1)
            def _():
                pltpu.make_async_copy(
                    ovs[b], out_hbm.at[pl.ds(0, _CH_OUT)], osems[b]).wait()

            x_v, out_v = xvs[b], ovs[b]

            @plsc.parallel_loop(0, _GROUPS, unroll=4)
            def _(i):
                sl = x_v.at[pl.ds(i * 64, 64)]
                a = plsc.load_gather(sl, [idx[0]])
                bb = plsc.load_gather(sl, [idx[1]])
                cc = plsc.load_gather(sl, [idx[2]])
                dd = plsc.load_gather(sl, [idx[3]])
                out_v[pl.ds(i * 16, 16)] = (a + bb + cc + dd) * 0.25

            pltpu.async_copy(
                out_v, out_hbm.at[pl.ds(out_base + c * _CH_OUT, _CH_OUT)],
                osems[b])
        return carry

    lax.fori_loop(0, _N_CHUNKS // 2, pair_body, 0)
    for b in (0, 1):
        pltpu.make_async_copy(
            ovs[b], out_hbm.at[pl.ds(0, _CH_OUT)], osems[b]).wait()


def kernel(x):
    xf = x.reshape(_TOTAL_IN)
    mesh = plsc.VectorSubcoreMesh(core_axis_name="c", subcore_axis_name="s")
    out = pl.kernel(
        _pool_body,
        out_type=jax.ShapeDtypeStruct((_TOTAL_OUT,), jnp.float32),
        mesh=mesh,
        scratch_types=[
            pltpu.VMEM((_CH_IN,), jnp.float32),
            pltpu.VMEM((_CH_IN,), jnp.float32),
            pltpu.VMEM((_CH_OUT,), jnp.float32),
            pltpu.VMEM((_CH_OUT,), jnp.float32),
            pltpu.SemaphoreType.DMA,
            pltpu.SemaphoreType.DMA,
            pltpu.SemaphoreType.DMA,
            pltpu.SemaphoreType.DMA,
        ],
        compiler_params=pltpu.CompilerParams(needs_layout_passes=False),
    )(xf)
    return out.reshape(_B, _C, _N // _K)
